# baseline (speedup 1.0000x reference)
import functools

import jax
import jax.numpy as jnp
from jax.experimental import pallas as pl

_B, _N, _D = 128, 100, 16
_ALPHA = 0.2
_JB = 8
_NL = 128
_F1 = 32
_F2 = 8
_NJP = 104
_NE = _N * _F1


def _leaky(v):
    return jnp.maximum(v, _ALPHA * v)


def _mp_kernel(x_ref, W1aT_ref, W1b_ref, b1_ref, Isel_ref, Mmask_ref,
               ones_ref, Wbd_ref, b2s_ref, fnW1T_ref, fnb1_ref,
               fnW2T_ref, fnb2_ref, o_ref, *, bb):
    W1aT = W1aT_ref[...]
    W1b = W1b_ref[...]
    b1 = b1_ref[...]
    Isel = Isel_ref[...]
    Mmask = Mmask_ref[...]
    ones = ones_ref[...]
    Wbd = Wbd_ref[...]
    b2s = b2s_ref[...]
    fnW1T = fnW1T_ref[...]
    fnb1 = fnb1_ref[...]
    fnW2T = fnW2T_ref[...]
    fnb2 = fnb2_ref[...]

    nfull = _N // _JB
    rem = _N - nfull * _JB

    for b in range(bb):
        x2d = x_ref[b]
        xz = jnp.concatenate(
            [x2d, jnp.zeros((_NL - _N, _D), jnp.float32)], axis=0)
        x_T = xz.T
        P_T = jnp.dot(W1aT, x_T, preferred_element_type=jnp.float32) + b1
        xp = xz[:_NJP, :]
        Q = jnp.dot(xp, W1b, preferred_element_type=jnp.float32)
        T1m = jnp.dot(Isel, Q, preferred_element_type=jnp.float32) * Mmask
        Qb = jnp.dot(T1m, ones, preferred_element_type=jnp.float32)
        Ptile = jnp.dot(Mmask, P_T, preferred_element_type=jnp.float32)
        E_all = _leaky(Ptile + Qb)

        acc = jnp.zeros((_JB * _F2, _NL), jnp.float32)
        for jb in range(nfull):
            Es = E_all[jb * _JB * _F1:(jb + 1) * _JB * _F1, :]
            acc = acc + _leaky(
                jnp.dot(Wbd, Es, preferred_element_type=jnp.float32) + b2s)
        Es = E_all[nfull * _JB * _F1:, :]
        Hr = _leaky(jnp.dot(Wbd[:rem * _F2, :rem * _F1], Es,
                            preferred_element_type=jnp.float32)
                    + b2s[:rem * _F2, :])

        A_T = (acc.reshape(_JB, _F2, _NL).sum(axis=0)
               + Hr.reshape(rem, _F2, _NL).sum(axis=0))

        hin_T = jnp.concatenate([A_T, x_T], axis=0)
        h_T = _leaky(jnp.dot(fnW1T, hin_T,
                             preferred_element_type=jnp.float32) + fnb1)
        o_T = (jnp.dot(fnW2T, h_T, preferred_element_type=jnp.float32)
               + fnb2)
        o_ref[b] = o_T.T[:_N, :]


def kernel(x, fe_W1, fe_b1, fe_W2, fe_b2, fn_W1, fn_b1, fn_W2, fn_b2):
    bb = 4
    grid = (_B // bb,)
    f32 = jnp.float32
    W1aT = fe_W1[:_D, :].T
    W1b = fe_W1[_D:, :]
    Isel = jnp.repeat(jnp.eye(_NJP, dtype=f32)[:_N], _F1, axis=0)
    Mmask = jnp.tile(jnp.eye(_F1, dtype=f32), (_N, 1))
    ones = jnp.ones((_F1, _NL), f32)
    Wbd = jnp.kron(jnp.eye(_JB, dtype=f32), fe_W2.T)
    b2s = jnp.broadcast_to(jnp.tile(fe_b2, (_JB,))[:, None],
                           (_JB * _F2, _NL))
    bcast = lambda v: jnp.broadcast_to(v[:, None], (v.shape[0], _NL))
    wspec = lambda r, c: pl.BlockSpec((r, c), lambda i: (0, 0))
    out = pl.pallas_call(
        functools.partial(_mp_kernel, bb=bb),
        grid=grid,
        in_specs=[
            pl.BlockSpec((bb, _N, _D), lambda i: (i, 0, 0)),
            wspec(_F1, _D),
            wspec(_D, _F1),
            wspec(_F1, _NL),
            wspec(_NE, _NJP),
            wspec(_NE, _F1),
            wspec(_F1, _NL),
            wspec(_JB * _F2, _JB * _F1),
            wspec(_JB * _F2, _NL),
            wspec(32, 24),
            wspec(32, _NL),
            wspec(16, 32),
            wspec(16, _NL),
        ],
        out_specs=pl.BlockSpec((bb, _N, _D), lambda i: (i, 0, 0)),
        out_shape=jax.ShapeDtypeStruct((_B, _N, _D), jnp.float32),
    )(x, W1aT, W1b, bcast(fe_b1), Isel, Mmask, ones, Wbd, b2s,
      fn_W1.T, bcast(fn_b1), fn_W2.T, bcast(fn_b2))
    return out


# speedup vs baseline: 1.7455x; 1.7455x over previous
import functools

import jax
import jax.numpy as jnp
from jax.experimental import pallas as pl

_B, _N, _D = 128, 100, 16
_ALPHA = 0.2
_JB = 8
_NL = 128
_F1 = 32
_F2 = 8


def _leaky(v):
    return jnp.maximum(v, _ALPHA * v)


def _mp_kernel(x_ref, W1a_ref, W1b_ref, b1_ref, Wbd_ref, b2s_ref,
               fnW1T_ref, fnb1_ref, fnW2T_ref, fnb2_ref, o_ref, *, bb):
    W1a = W1a_ref[...]
    W1b = W1b_ref[...]
    b1 = b1_ref[0, :]
    Wbd = Wbd_ref[...]
    b2s = b2s_ref[...]
    fnW1T = fnW1T_ref[...]
    fnb1 = fnb1_ref[...]
    fnW2T = fnW2T_ref[...]
    fnb2 = fnb2_ref[...]

    nfull = _N // _JB
    rem = _N - nfull * _JB

    for b in range(bb):
        x2d = x_ref[b]
        P = jnp.dot(x2d, W1a, preferred_element_type=jnp.float32) + b1
        Q = jnp.dot(x2d, W1b, preferred_element_type=jnp.float32)
        Pz = jnp.concatenate(
            [P, jnp.zeros((_NL - _N, _F1), jnp.float32)], axis=0)
        P_T = Pz.T

        acc = jnp.zeros((_JB * _F2, _NL), jnp.float32)
        for jb in range(nfull):
            Qblk = Q[jb * _JB:(jb + 1) * _JB, :]
            Qb3 = jnp.broadcast_to(Qblk[:, :, None], (_JB, _F1, _NL))
            E = _leaky(P_T[None, :, :] + Qb3)
            Es = E.reshape(_JB * _F1, _NL)
            Hs = _leaky(jnp.dot(Wbd, Es, preferred_element_type=jnp.float32)
                        + b2s)
            acc = acc + Hs
        Qblk = Q[nfull * _JB:_N, :]
        Qb3 = jnp.broadcast_to(Qblk[:, :, None], (rem, _F1, _NL))
        E = _leaky(P_T[None, :, :] + Qb3)
        Es = E.reshape(rem * _F1, _NL)
        Hr = _leaky(jnp.dot(Wbd[:rem * _F2, :rem * _F1], Es,
                            preferred_element_type=jnp.float32)
                    + b2s[:rem * _F2, :])

        A_T = (acc.reshape(_JB, _F2, _NL).sum(axis=0)
               + Hr.reshape(rem, _F2, _NL).sum(axis=0))

        xz = jnp.concatenate(
            [x2d, jnp.zeros((_NL - _N, _D), jnp.float32)], axis=0)
        x_T = xz.T
        hin_T = jnp.concatenate([A_T, x_T], axis=0)
        h_T = _leaky(jnp.dot(fnW1T, hin_T,
                             preferred_element_type=jnp.float32) + fnb1)
        o_T = (jnp.dot(fnW2T, h_T, preferred_element_type=jnp.float32)
               + fnb2)
        o_ref[b] = o_T.T[:_N, :]


def kernel(x, fe_W1, fe_b1, fe_W2, fe_b2, fn_W1, fn_b1, fn_W2, fn_b2):
    bb = 8
    grid = (_B // bb,)
    W1a = fe_W1[:_D, :]
    W1b = fe_W1[_D:, :]
    Wbd = jnp.kron(jnp.eye(_JB, dtype=jnp.float32), fe_W2.T)
    b2s = jnp.tile(fe_b2, (_JB,)).reshape(_JB * _F2, 1)
    wspec = lambda r, c: pl.BlockSpec((r, c), lambda i: (0, 0))
    out = pl.pallas_call(
        functools.partial(_mp_kernel, bb=bb),
        grid=grid,
        in_specs=[
            pl.BlockSpec((bb, _N, _D), lambda i: (i, 0, 0)),
            wspec(_D, _F1),
            wspec(_D, _F1),
            wspec(1, _F1),
            wspec(_JB * _F2, _JB * _F1),
            wspec(_JB * _F2, 1),
            wspec(32, 24),
            wspec(32, 1),
            wspec(16, 32),
            wspec(16, 1),
        ],
        out_specs=pl.BlockSpec((bb, _N, _D), lambda i: (i, 0, 0)),
        out_shape=jax.ShapeDtypeStruct((_B, _N, _D), jnp.float32),
    )(x, W1a, W1b, fe_b1.reshape(1, -1), Wbd, b2s,
      fn_W1.T, fn_b1.reshape(-1, 1), fn_W2.T, fn_b2.reshape(-1, 1))
    return out


# speedup vs baseline: 2.0793x; 1.1912x over previous
import functools

import jax
import jax.numpy as jnp
from jax.experimental import pallas as pl

_B, _N, _D = 128, 100, 16
_ALPHA = 0.2
_JB = 8
_NL = 128
_F1 = 32
_F2 = 8


def _leaky(v):
    return jnp.maximum(v, _ALPHA * v)


def _mp_kernel(x_ref, W1a_ref, W1b_ref, b1_ref, Wbd_ref, b2s_ref,
               fnW1T_ref, fnb1_ref, fnW2T_ref, fnb2_ref, o_ref, *, bb):
    W1a = W1a_ref[...]
    W1b = W1b_ref[...]
    b1 = b1_ref[0, :]
    Wbd = Wbd_ref[...]
    b2s = b2s_ref[...]
    fnW1T = fnW1T_ref[...]
    fnb1 = fnb1_ref[...]
    fnW2T = fnW2T_ref[...]
    fnb2 = fnb2_ref[...]

    nfull = _N // _JB
    rem = _N - nfull * _JB

    for b in range(bb):
        x2d = x_ref[b]
        P = jnp.dot(x2d, W1a, preferred_element_type=jnp.float32) + b1
        Q = (jnp.dot(x2d, W1b, preferred_element_type=jnp.float32)
             .astype(jnp.bfloat16))
        Pz = jnp.concatenate(
            [P, jnp.zeros((_NL - _N, _F1), jnp.float32)], axis=0)
        P_T = Pz.T.astype(jnp.bfloat16)
        Wbd_b = Wbd.astype(jnp.bfloat16)

        acc = jnp.zeros((_JB * _F2, _NL), jnp.float32)
        for jb in range(nfull):
            Qblk = Q[jb * _JB:(jb + 1) * _JB, :]
            Qb3 = jnp.broadcast_to(Qblk[:, :, None], (_JB, _F1, _NL))
            E = _leaky(P_T[None, :, :] + Qb3)
            Es = E.reshape(_JB * _F1, _NL)
            Hs = _leaky(jnp.dot(Wbd_b, Es,
                                preferred_element_type=jnp.float32)
                        + b2s)
            acc = acc + Hs
        Qblk = Q[nfull * _JB:_N, :]
        Qb3 = jnp.broadcast_to(Qblk[:, :, None], (rem, _F1, _NL))
        E = _leaky(P_T[None, :, :] + Qb3)
        Es = E.reshape(rem * _F1, _NL)
        Hr = _leaky(jnp.dot(Wbd_b[:rem * _F2, :rem * _F1], Es,
                            preferred_element_type=jnp.float32)
                    + b2s[:rem * _F2, :])

        A_T = (acc.reshape(_JB, _F2, _NL).sum(axis=0)
               + Hr.reshape(rem, _F2, _NL).sum(axis=0))

        xz = jnp.concatenate(
            [x2d, jnp.zeros((_NL - _N, _D), jnp.float32)], axis=0)
        x_T = xz.T
        hin_T = jnp.concatenate([A_T, x_T], axis=0)
        h_T = _leaky(jnp.dot(fnW1T, hin_T,
                             preferred_element_type=jnp.float32) + fnb1)
        o_T = (jnp.dot(fnW2T, h_T, preferred_element_type=jnp.float32)
               + fnb2)
        o_ref[b] = o_T.T[:_N, :]


def kernel(x, fe_W1, fe_b1, fe_W2, fe_b2, fn_W1, fn_b1, fn_W2, fn_b2):
    bb = 8
    grid = (_B // bb,)
    W1a = fe_W1[:_D, :]
    W1b = fe_W1[_D:, :]
    Wbd = jnp.kron(jnp.eye(_JB, dtype=jnp.float32), fe_W2.T)
    b2s = jnp.tile(fe_b2, (_JB,)).reshape(_JB * _F2, 1)
    wspec = lambda r, c: pl.BlockSpec((r, c), lambda i: (0, 0))
    out = pl.pallas_call(
        functools.partial(_mp_kernel, bb=bb),
        grid=grid,
        in_specs=[
            pl.BlockSpec((bb, _N, _D), lambda i: (i, 0, 0)),
            wspec(_D, _F1),
            wspec(_D, _F1),
            wspec(1, _F1),
            wspec(_JB * _F2, _JB * _F1),
            wspec(_JB * _F2, 1),
            wspec(32, 24),
            wspec(32, 1),
            wspec(16, 32),
            wspec(16, 1),
        ],
        out_specs=pl.BlockSpec((bb, _N, _D), lambda i: (i, 0, 0)),
        out_shape=jax.ShapeDtypeStruct((_B, _N, _D), jnp.float32),
    )(x, W1a, W1b, fe_b1.reshape(1, -1), Wbd, b2s,
      fn_W1.T, fn_b1.reshape(-1, 1), fn_W2.T, fn_b2.reshape(-1, 1))
    return out


# speedup vs baseline: 3.5696x; 1.7167x over previous
import functools

import jax
import jax.numpy as jnp
from jax.experimental import pallas as pl

_B, _N, _D = 128, 100, 16
_ALPHA = 0.2
_JB = 8
_NL = 128
_F1 = 32
_F2 = 8


def _leaky(v):
    return jnp.maximum(v, _ALPHA * v)


def _mp_kernel(x_ref, W1a_ref, W1b_ref, b1_ref, Wbd_ref, b2s_ref,
               fnW1T_ref, fnb1_ref, fnW2T_ref, fnb2_ref, o_ref, *, bb):
    W1a = W1a_ref[...]
    W1b = W1b_ref[...]
    b1 = b1_ref[0, :]
    Wbd = Wbd_ref[...]
    b2s = b2s_ref[...]
    fnW1T = fnW1T_ref[...]
    fnb1 = fnb1_ref[...]
    fnW2T = fnW2T_ref[...]
    fnb2 = fnb2_ref[...]

    nfull = _N // _JB
    rem = _N - nfull * _JB
    Wbd_b = Wbd.astype(jnp.bfloat16)

    xTs, PTs, Qs = [], [], []
    for b in range(bb):
        x2d = x_ref[b]
        xz = jnp.concatenate(
            [x2d, jnp.zeros((_NL - _N, _D), jnp.float32)], axis=0)
        x_T = xz.T
        P = jnp.dot(x2d, W1a, preferred_element_type=jnp.float32) + b1
        Q = (jnp.dot(x2d, W1b, preferred_element_type=jnp.float32)
             .astype(jnp.bfloat16))
        Pz = jnp.concatenate(
            [P, jnp.zeros((_NL - _N, _F1), jnp.float32)], axis=0)
        xTs.append(x_T)
        PTs.append(Pz.T.astype(jnp.bfloat16))
        Qs.append(Q)

    accs = [jnp.zeros((_JB * _F2, _NL), jnp.float32) for _ in range(bb)]
    for jb in range(nfull):
        for b in range(bb):
            Qblk = Qs[b][jb * _JB:(jb + 1) * _JB, :]
            Qb3 = jnp.broadcast_to(Qblk[:, :, None], (_JB, _F1, _NL))
            E = _leaky(PTs[b][None, :, :] + Qb3)
            Es = E.reshape(_JB * _F1, _NL)
            Hs = _leaky(jnp.dot(Wbd_b, Es,
                                preferred_element_type=jnp.float32)
                        + b2s)
            accs[b] = accs[b] + Hs
    ATs = []
    for b in range(bb):
        Qblk = Qs[b][nfull * _JB:_N, :]
        Qb3 = jnp.broadcast_to(Qblk[:, :, None], (rem, _F1, _NL))
        E = _leaky(PTs[b][None, :, :] + Qb3)
        Es = E.reshape(rem * _F1, _NL)
        Hr = _leaky(jnp.dot(Wbd_b[:rem * _F2, :rem * _F1], Es,
                            preferred_element_type=jnp.float32)
                    + b2s[:rem * _F2, :])
        ATs.append(accs[b].reshape(_JB, _F2, _NL).sum(axis=0)
                   + Hr.reshape(rem, _F2, _NL).sum(axis=0))

    hin = jnp.concatenate(
        [jnp.concatenate([ATs[b], xTs[b]], axis=0) for b in range(bb)],
        axis=1)
    h = _leaky(jnp.dot(fnW1T, hin,
                       preferred_element_type=jnp.float32) + fnb1)
    o = (jnp.dot(fnW2T, h, preferred_element_type=jnp.float32)
         + fnb2)
    oT = o.T
    for b in range(bb):
        o_ref[b] = oT[b * _NL:b * _NL + _N, :]


def kernel(x, fe_W1, fe_b1, fe_W2, fe_b2, fn_W1, fn_b1, fn_W2, fn_b2):
    bb = 8
    grid = (_B // bb,)
    W1a = fe_W1[:_D, :]
    W1b = fe_W1[_D:, :]
    Wbd = jnp.kron(jnp.eye(_JB, dtype=jnp.float32), fe_W2.T)
    b2s = jnp.tile(fe_b2, (_JB,)).reshape(_JB * _F2, 1)
    wspec = lambda r, c: pl.BlockSpec((r, c), lambda i: (0, 0))
    out = pl.pallas_call(
        functools.partial(_mp_kernel, bb=bb),
        grid=grid,
        in_specs=[
            pl.BlockSpec((bb, _N, _D), lambda i: (i, 0, 0)),
            wspec(_D, _F1),
            wspec(_D, _F1),
            wspec(1, _F1),
            wspec(_JB * _F2, _JB * _F1),
            wspec(_JB * _F2, 1),
            wspec(32, 24),
            wspec(32, 1),
            wspec(16, 32),
            wspec(16, 1),
        ],
        out_specs=pl.BlockSpec((bb, _N, _D), lambda i: (i, 0, 0)),
        out_shape=jax.ShapeDtypeStruct((_B, _N, _D), jnp.float32),
    )(x, W1a, W1b, fe_b1.reshape(1, -1), Wbd, b2s,
      fn_W1.T, fn_b1.reshape(-1, 1), fn_W2.T, fn_b2.reshape(-1, 1))
    return out


# speedup vs baseline: 4.0105x; 1.1235x over previous
import functools

import jax
import jax.numpy as jnp
from jax.experimental import pallas as pl

_B, _N, _D = 128, 100, 16
_ALPHA = 0.2
_JB = 8
_NL = 128
_F1 = 32
_F2 = 8


def _leaky(v):
    return jnp.maximum(v, _ALPHA * v)


def _mp_kernel(x_ref, W1a_ref, W1b_ref, b1_ref, Wbd_ref, b2s_ref,
               fnW1T_ref, fnb1_ref, fnW2T_ref, fnb2_ref, o_ref, *, bb):
    W1a = W1a_ref[...]
    W1b = W1b_ref[...]
    b1 = b1_ref[0, :]
    Wbd = Wbd_ref[...]
    b2s = b2s_ref[...]
    fnW1T = fnW1T_ref[...]
    fnb1 = fnb1_ref[...]
    fnW2T = fnW2T_ref[...]
    fnb2 = fnb2_ref[...]

    nfull = _N // _JB
    rem = _N - nfull * _JB
    Wbd_b = Wbd.astype(jnp.bfloat16)

    xTs, PTs, Qs = [], [], []
    for b in range(bb):
        x2d = x_ref[b]
        xz = jnp.concatenate(
            [x2d, jnp.zeros((_NL - _N, _D), jnp.float32)], axis=0)
        x_T = xz.T
        P = jnp.dot(x2d, W1a, preferred_element_type=jnp.float32) + b1
        Q = (jnp.dot(x2d, W1b, preferred_element_type=jnp.float32)
             .astype(jnp.bfloat16))
        Pz = jnp.concatenate(
            [P, jnp.zeros((_NL - _N, _F1), jnp.float32)], axis=0)
        xTs.append(x_T)
        PTs.append(Pz.T.astype(jnp.bfloat16))
        Qs.append(Q)

    accs = [jnp.zeros((_JB * _F2, _NL), jnp.float32) for _ in range(bb)]
    for jb in range(nfull):
        for b in range(bb):
            Qblk = Qs[b][jb * _JB:(jb + 1) * _JB, :]
            Qb3 = jnp.broadcast_to(Qblk[:, :, None], (_JB, _F1, _NL))
            E = _leaky(PTs[b][None, :, :] + Qb3)
            Es = E.reshape(_JB * _F1, _NL)
            Hs = _leaky(jnp.dot(Wbd_b, Es,
                                preferred_element_type=jnp.float32)
                        + b2s)
            accs[b] = accs[b] + Hs
    ATs = []
    for b in range(bb):
        Qblk = Qs[b][nfull * _JB:_N, :]
        Qb3 = jnp.broadcast_to(Qblk[:, :, None], (rem, _F1, _NL))
        E = _leaky(PTs[b][None, :, :] + Qb3)
        Es = E.reshape(rem * _F1, _NL)
        Hr = _leaky(jnp.dot(Wbd_b[:rem * _F2, :rem * _F1], Es,
                            preferred_element_type=jnp.float32)
                    + b2s[:rem * _F2, :])
        ATs.append(accs[b].reshape(_JB, _F2, _NL).sum(axis=0)
                   + Hr.reshape(rem, _F2, _NL).sum(axis=0))

    hin = jnp.concatenate(
        [jnp.concatenate([ATs[b], xTs[b]], axis=0) for b in range(bb)],
        axis=1)
    h = _leaky(jnp.dot(fnW1T, hin,
                       preferred_element_type=jnp.float32) + fnb1)
    o = (jnp.dot(fnW2T, h, preferred_element_type=jnp.float32)
         + fnb2)
    oT = o.T
    for b in range(bb):
        o_ref[b] = oT[b * _NL:b * _NL + _N, :]


def kernel(x, fe_W1, fe_b1, fe_W2, fe_b2, fn_W1, fn_b1, fn_W2, fn_b2):
    bb = 16
    grid = (_B // bb,)
    W1a = fe_W1[:_D, :]
    W1b = fe_W1[_D:, :]
    Wbd = jnp.kron(jnp.eye(_JB, dtype=jnp.float32), fe_W2.T)
    b2s = jnp.tile(fe_b2, (_JB,)).reshape(_JB * _F2, 1)
    wspec = lambda r, c: pl.BlockSpec((r, c), lambda i: (0, 0))
    out = pl.pallas_call(
        functools.partial(_mp_kernel, bb=bb),
        grid=grid,
        in_specs=[
            pl.BlockSpec((bb, _N, _D), lambda i: (i, 0, 0)),
            wspec(_D, _F1),
            wspec(_D, _F1),
            wspec(1, _F1),
            wspec(_JB * _F2, _JB * _F1),
            wspec(_JB * _F2, 1),
            wspec(32, 24),
            wspec(32, 1),
            wspec(16, 32),
            wspec(16, 1),
        ],
        out_specs=pl.BlockSpec((bb, _N, _D), lambda i: (i, 0, 0)),
        out_shape=jax.ShapeDtypeStruct((_B, _N, _D), jnp.float32),
    )(x, W1a, W1b, fe_b1.reshape(1, -1), Wbd, b2s,
      fn_W1.T, fn_b1.reshape(-1, 1), fn_W2.T, fn_b2.reshape(-1, 1))
    return out


# speedup vs baseline: 4.1910x; 1.0450x over previous
import functools

import jax
import jax.numpy as jnp
from jax.experimental import pallas as pl

_B, _N, _D = 128, 100, 16
_ALPHA = 0.2
_JB = 8
_NL = 128
_F1 = 32
_F2 = 8


def _leaky(v):
    return jnp.maximum(v, _ALPHA * v)


def _mp_kernel(x_ref, W1a_ref, W1b_ref, b1_ref, Wbd_ref, b2s_ref,
               fnW1T_ref, fnb1_ref, fnW2T_ref, fnb2_ref, o_ref, *, bb):
    W1a = W1a_ref[...]
    W1b = W1b_ref[...]
    b1 = b1_ref[0, :]
    Wbd = Wbd_ref[...]
    b2s = b2s_ref[...]
    fnW1T = fnW1T_ref[...]
    fnb1 = fnb1_ref[...]
    fnW2T = fnW2T_ref[...]
    fnb2 = fnb2_ref[...]

    nfull = _N // _JB
    rem = _N - nfull * _JB
    Wbd_b = Wbd.astype(jnp.bfloat16)

    xTs, PTs, Qs = [], [], []
    for b in range(bb):
        x2d = x_ref[b]
        xz = jnp.concatenate(
            [x2d, jnp.zeros((_NL - _N, _D), jnp.float32)], axis=0)
        x_T = xz.T
        P = jnp.dot(x2d, W1a, preferred_element_type=jnp.float32) + b1
        Q = (jnp.dot(x2d, W1b, preferred_element_type=jnp.float32)
             .astype(jnp.bfloat16))
        Pz = jnp.concatenate(
            [P, jnp.zeros((_NL - _N, _F1), jnp.float32)], axis=0)
        xTs.append(x_T)
        PTs.append(Pz.T.astype(jnp.bfloat16))
        Qs.append(Q)

    accs = [jnp.zeros((_JB * _F2, _NL), jnp.float32) for _ in range(bb)]
    for jb in range(nfull):
        for b in range(bb):
            Qblk = Qs[b][jb * _JB:(jb + 1) * _JB, :]
            Qb3 = jnp.broadcast_to(Qblk[:, :, None], (_JB, _F1, _NL))
            E = _leaky(PTs[b][None, :, :] + Qb3)
            Es = E.reshape(_JB * _F1, _NL)
            Hs = _leaky(jnp.dot(Wbd_b, Es,
                                preferred_element_type=jnp.float32)
                        + b2s)
            accs[b] = accs[b] + Hs
    ATs = []
    for b in range(bb):
        Qblk = Qs[b][nfull * _JB:_N, :]
        Qb3 = jnp.broadcast_to(Qblk[:, :, None], (rem, _F1, _NL))
        E = _leaky(PTs[b][None, :, :] + Qb3)
        Es = E.reshape(rem * _F1, _NL)
        Hr = _leaky(jnp.dot(Wbd_b[:rem * _F2, :rem * _F1], Es,
                            preferred_element_type=jnp.float32)
                    + b2s[:rem * _F2, :])
        ATs.append(accs[b].reshape(_JB, _F2, _NL).sum(axis=0)
                   + Hr.reshape(rem, _F2, _NL).sum(axis=0))

    hin = jnp.concatenate(
        [jnp.concatenate([ATs[b], xTs[b]], axis=0) for b in range(bb)],
        axis=1)
    h = _leaky(jnp.dot(fnW1T, hin,
                       preferred_element_type=jnp.float32) + fnb1)
    o = (jnp.dot(fnW2T, h, preferred_element_type=jnp.float32)
         + fnb2)
    oT = o.T
    for b in range(bb):
        o_ref[b] = oT[b * _NL:b * _NL + _N, :]


def kernel(x, fe_W1, fe_b1, fe_W2, fe_b2, fn_W1, fn_b1, fn_W2, fn_b2):
    bb = 32
    grid = (_B // bb,)
    W1a = fe_W1[:_D, :]
    W1b = fe_W1[_D:, :]
    Wbd = jnp.kron(jnp.eye(_JB, dtype=jnp.float32), fe_W2.T)
    b2s = jnp.tile(fe_b2, (_JB,)).reshape(_JB * _F2, 1)
    wspec = lambda r, c: pl.BlockSpec((r, c), lambda i: (0, 0))
    out = pl.pallas_call(
        functools.partial(_mp_kernel, bb=bb),
        grid=grid,
        in_specs=[
            pl.BlockSpec((bb, _N, _D), lambda i: (i, 0, 0)),
            wspec(_D, _F1),
            wspec(_D, _F1),
            wspec(1, _F1),
            wspec(_JB * _F2, _JB * _F1),
            wspec(_JB * _F2, 1),
            wspec(32, 24),
            wspec(32, 1),
            wspec(16, 32),
            wspec(16, 1),
        ],
        out_specs=pl.BlockSpec((bb, _N, _D), lambda i: (i, 0, 0)),
        out_shape=jax.ShapeDtypeStruct((_B, _N, _D), jnp.float32),
    )(x, W1a, W1b, fe_b1.reshape(1, -1), Wbd, b2s,
      fn_W1.T, fn_b1.reshape(-1, 1), fn_W2.T, fn_b2.reshape(-1, 1))
    return out


# speedup vs baseline: 4.5483x; 1.0853x over previous
import functools

import jax
import jax.numpy as jnp
from jax.experimental import pallas as pl

_B, _N, _D = 128, 100, 16
_ALPHA = 0.2
_JB = 8
_NL = 128
_F1 = 32
_F2 = 8


def _leaky(v):
    return jnp.maximum(v, _ALPHA * v)


def _mp_kernel(x_ref, feW1_ref, feW2_ref, fnW1_ref, fnW2_ref, bias_ref,
               o_ref, *, bb):
    W1a = feW1_ref[:_D, :]
    W1b = feW1_ref[_D:, :]
    b1 = bias_ref[0, 0:_F1]
    W2T = feW2_ref[...].T
    tiles = jnp.concatenate([W2T] * _JB, axis=1)
    tiles = jnp.concatenate([tiles] * _JB, axis=0)
    rowg = jax.lax.broadcasted_iota(jnp.int32, (_JB * _F2, _JB * _F1), 0)
    colg = jax.lax.broadcasted_iota(jnp.int32, (_JB * _F2, _JB * _F1), 1)
    Wbd_b = jnp.where(rowg // _F2 == colg // _F1, tiles,
                      0.0).astype(jnp.bfloat16)
    b2 = bias_ref[0, _F1:_F1 + _F2]
    b2s = jnp.concatenate([b2] * _JB, axis=0).reshape(_JB * _F2, 1)
    fnW1T = fnW1_ref[...].T
    fnb1 = bias_ref[0, _F1 + _F2:_F1 + _F2 + 32].reshape(32, 1)
    fnW2T = fnW2_ref[...].T
    fnb2 = bias_ref[0, _F1 + _F2 + 32:].reshape(16, 1)

    nfull = _N // _JB
    rem = _N - nfull * _JB

    xTs, PTs, Qs = [], [], []
    for b in range(bb):
        x2d = x_ref[b]
        xz = jnp.concatenate(
            [x2d, jnp.zeros((_NL - _N, _D), jnp.float32)], axis=0)
        x_T = xz.T
        P = jnp.dot(x2d, W1a, preferred_element_type=jnp.float32) + b1
        Q = (jnp.dot(x2d, W1b, preferred_element_type=jnp.float32)
             .astype(jnp.bfloat16))
        Pz = jnp.concatenate(
            [P, jnp.zeros((_NL - _N, _F1), jnp.float32)], axis=0)
        xTs.append(x_T)
        PTs.append(Pz.T.astype(jnp.bfloat16))
        Qs.append(Q)

    accs = [jnp.zeros((_JB * _F2, _NL), jnp.float32) for _ in range(bb)]
    for jb in range(nfull):
        for b in range(bb):
            Qblk = Qs[b][jb * _JB:(jb + 1) * _JB, :]
            Qb3 = jnp.broadcast_to(Qblk[:, :, None], (_JB, _F1, _NL))
            E = _leaky(PTs[b][None, :, :] + Qb3)
            Es = E.reshape(_JB * _F1, _NL)
            Hs = _leaky(jnp.dot(Wbd_b, Es,
                                preferred_element_type=jnp.float32)
                        + b2s)
            accs[b] = accs[b] + Hs
    ATs = []
    for b in range(bb):
        Qblk = Qs[b][nfull * _JB:_N, :]
        Qb3 = jnp.broadcast_to(Qblk[:, :, None], (rem, _F1, _NL))
        E = _leaky(PTs[b][None, :, :] + Qb3)
        Es = E.reshape(rem * _F1, _NL)
        Hr = _leaky(jnp.dot(Wbd_b[:rem * _F2, :rem * _F1], Es,
                            preferred_element_type=jnp.float32)
                    + b2s[:rem * _F2, :])
        ATs.append(accs[b].reshape(_JB, _F2, _NL).sum(axis=0)
                   + Hr.reshape(rem, _F2, _NL).sum(axis=0))

    hin = jnp.concatenate(
        [jnp.concatenate([ATs[b], xTs[b]], axis=0) for b in range(bb)],
        axis=1)
    h = _leaky(jnp.dot(fnW1T, hin,
                       preferred_element_type=jnp.float32) + fnb1)
    o = (jnp.dot(fnW2T, h, preferred_element_type=jnp.float32)
         + fnb2)
    oT = o.T
    for b in range(bb):
        o_ref[b] = oT[b * _NL:b * _NL + _N, :]


def kernel(x, fe_W1, fe_b1, fe_W2, fe_b2, fn_W1, fn_b1, fn_W2, fn_b2):
    bb = 32
    grid = (_B // bb,)
    bias = jnp.concatenate([fe_b1, fe_b2, fn_b1, fn_b2]).reshape(1, -1)
    wspec = lambda r, c: pl.BlockSpec((r, c), lambda i: (0, 0))
    out = pl.pallas_call(
        functools.partial(_mp_kernel, bb=bb),
        grid=grid,
        in_specs=[
            pl.BlockSpec((bb, _N, _D), lambda i: (i, 0, 0)),
            wspec(2 * _D, _F1),
            wspec(_F1, _F2),
            wspec(24, 32),
            wspec(32, 16),
            wspec(1, 88),
        ],
        out_specs=pl.BlockSpec((bb, _N, _D), lambda i: (i, 0, 0)),
        out_shape=jax.ShapeDtypeStruct((_B, _N, _D), jnp.float32),
    )(x, fe_W1, fe_W2, fn_W1, fn_W2, bias)
    return out


# speedup vs baseline: 4.5537x; 1.0012x over previous
import functools

import jax
import jax.numpy as jnp
from jax.experimental import pallas as pl

_B, _N, _D = 128, 100, 16
_ALPHA = 0.2
_JB = 8
_NL = 128
_F1 = 32
_F2 = 8


def _leaky(v):
    return jnp.maximum(v, _ALPHA * v)


def _mp_kernel(x_ref, feW1_ref, feW2_ref, fnW1_ref, fnW2_ref, bias_ref,
               o_ref, *, bb):
    W1a = feW1_ref[:_D, :]
    W1b = feW1_ref[_D:, :]
    b1 = bias_ref[0, 0:_F1]
    W2T = feW2_ref[...].T
    tiles = jnp.concatenate([W2T] * _JB, axis=1)
    tiles = jnp.concatenate([tiles] * _JB, axis=0)
    rowg = jax.lax.broadcasted_iota(jnp.int32, (_JB * _F2, _JB * _F1), 0)
    colg = jax.lax.broadcasted_iota(jnp.int32, (_JB * _F2, _JB * _F1), 1)
    Wbd_b = jnp.where(rowg // _F2 == colg // _F1, tiles,
                      0.0).astype(jnp.bfloat16)
    b2 = bias_ref[0, _F1:_F1 + _F2]
    b2s = jnp.concatenate([b2] * _JB, axis=0).reshape(_JB * _F2, 1)
    fnW1T = fnW1_ref[...].T
    fnb1 = bias_ref[0, _F1 + _F2:_F1 + _F2 + 32].reshape(32, 1)
    fnW2T = fnW2_ref[...].T
    fnb2 = bias_ref[0, _F1 + _F2 + 32:].reshape(16, 1)

    nfull = _N // _JB
    rem = _N - nfull * _JB

    xTs, PTs, Qs = [], [], []
    for b in range(bb):
        x2d = x_ref[b]
        xz = jnp.concatenate(
            [x2d, jnp.zeros((_NL - _N, _D), jnp.float32)], axis=0)
        x_T = xz.T
        P = jnp.dot(x2d, W1a, preferred_element_type=jnp.float32) + b1
        Q = (jnp.dot(x2d, W1b, preferred_element_type=jnp.float32)
             .astype(jnp.bfloat16))
        Pz = jnp.concatenate(
            [P, jnp.zeros((_NL - _N, _F1), jnp.float32)], axis=0)
        xTs.append(x_T)
        PTs.append(Pz.T.astype(jnp.bfloat16))
        Qs.append(Q)

    accs = [jnp.zeros((_JB * _F2, _NL), jnp.float32) for _ in range(bb)]
    for jb in range(nfull):
        for b in range(bb):
            Qblk = Qs[b][jb * _JB:(jb + 1) * _JB, :]
            Qb3 = jnp.broadcast_to(Qblk[:, :, None], (_JB, _F1, _NL))
            E = _leaky(PTs[b][None, :, :] + Qb3)
            Es = E.reshape(_JB * _F1, _NL)
            Hs = _leaky(jnp.dot(Wbd_b, Es,
                                preferred_element_type=jnp.float32)
                        + b2s)
            accs[b] = accs[b] + Hs
    ATs = []
    for b in range(bb):
        Qblk = Qs[b][nfull * _JB:_N, :]
        Qb3 = jnp.broadcast_to(Qblk[:, :, None], (rem, _F1, _NL))
        E = _leaky(PTs[b][None, :, :] + Qb3)
        Es = E.reshape(rem * _F1, _NL)
        Hr = _leaky(jnp.dot(Wbd_b[:rem * _F2, :rem * _F1], Es,
                            preferred_element_type=jnp.float32)
                    + b2s[:rem * _F2, :])
        ATs.append(accs[b].reshape(_JB, _F2, _NL).sum(axis=0)
                   + Hr.reshape(rem, _F2, _NL).sum(axis=0))

    hin = jnp.concatenate(
        [jnp.concatenate([ATs[b], xTs[b]], axis=0) for b in range(bb)],
        axis=1)
    h = _leaky(jnp.dot(fnW1T, hin,
                       preferred_element_type=jnp.float32) + fnb1)
    o = (jnp.dot(fnW2T, h, preferred_element_type=jnp.float32)
         + fnb2)
    oT = o.T
    for b in range(bb):
        o_ref[b] = oT[b * _NL:b * _NL + _N, :]


def kernel(x, fe_W1, fe_b1, fe_W2, fe_b2, fn_W1, fn_b1, fn_W2, fn_b2):
    bb = 64
    grid = (_B // bb,)
    bias = jnp.concatenate([fe_b1, fe_b2, fn_b1, fn_b2]).reshape(1, -1)
    wspec = lambda r, c: pl.BlockSpec((r, c), lambda i: (0, 0))
    out = pl.pallas_call(
        functools.partial(_mp_kernel, bb=bb),
        grid=grid,
        in_specs=[
            pl.BlockSpec((bb, _N, _D), lambda i: (i, 0, 0)),
            wspec(2 * _D, _F1),
            wspec(_F1, _F2),
            wspec(24, 32),
            wspec(32, 16),
            wspec(1, 88),
        ],
        out_specs=pl.BlockSpec((bb, _N, _D), lambda i: (i, 0, 0)),
        out_shape=jax.ShapeDtypeStruct((_B, _N, _D), jnp.float32),
    )(x, fe_W1, fe_W2, fn_W1, fn_W2, bias)
    return out
